# Initial kernel scaffold; baseline (speedup 1.0000x reference)
import jax
import jax.numpy as jnp
from jax.experimental import pallas as pl


def kernel(x, Wr, br, W1, b1, W2, b2):
    raise NotImplementedError("write your pallas kernel here")


# speedup vs baseline: 1.1794x; 1.1794x over previous
import functools

import jax
import jax.numpy as jnp
from jax.experimental import pallas as pl

D = 768
H = 3072
E = 8
K = 2
S = 2048

TOK_BLK = 512


def _router_body(x_ref, wr_ref, br_ref, gate_ref):
    logits = jnp.dot(x_ref[...], wr_ref[...].T,
                     preferred_element_type=jnp.float32) + br_ref[...]
    neg = jnp.float32(-jnp.inf)
    best1 = logits[:, 0:1]
    idx1 = jnp.zeros_like(best1, dtype=jnp.int32)
    for e in range(1, E):
        v = logits[:, e:e + 1]
        upd = v > best1
        best1 = jnp.where(upd, v, best1)
        idx1 = jnp.where(upd, e, idx1)
    best2 = jnp.full_like(best1, neg)
    idx2 = jnp.zeros_like(best1, dtype=jnp.int32)
    for e in range(E):
        v = logits[:, e:e + 1]
        v = jnp.where(idx1 == e, neg, v)
        upd = v > best2
        best2 = jnp.where(upd, v, best2)
        idx2 = jnp.where(upd, e, idx2)
    m = jnp.maximum(best1, best2)
    e1 = jnp.exp(best1 - m)
    e2 = jnp.exp(best2 - m)
    denom = e1 + e2
    w1 = e1 / denom
    w2 = e2 / denom
    cols = jax.lax.broadcasted_iota(jnp.int32, (S, E), 1)
    gate = jnp.where(cols == idx1, w1, 0.0) + jnp.where(cols == idx2, w2, 0.0)
    gate_ref[...] = gate


def _expert_body(gate_ref, x_ref, w1_ref, b1_ref, w2_ref, b2_ref, out_ref):
    e = pl.program_id(1)

    @pl.when(e == 0)
    def _init():
        out_ref[...] = jnp.zeros_like(out_ref)

    h = jnp.dot(x_ref[...], w1_ref[0].T,
                preferred_element_type=jnp.float32) + b1_ref[0, 0]
    h = jnp.maximum(h, 0.0)
    y = jnp.dot(h, w2_ref[0].T,
                preferred_element_type=jnp.float32) + b2_ref[0, 0]
    cols = jax.lax.broadcasted_iota(jnp.int32, (TOK_BLK, E), 1)
    g = jnp.sum(jnp.where(cols == e, gate_ref[...], 0.0), axis=1,
                keepdims=True)
    out_ref[...] += g * y


@jax.jit
def kernel(x, Wr, br, W1, b1, W2, b2):
    xs = x.reshape(S, D)
    gate = pl.pallas_call(
        _router_body,
        out_shape=jax.ShapeDtypeStruct((S, E), jnp.float32),
    )(xs, Wr, br)

    n_tok = S // TOK_BLK
    out = pl.pallas_call(
        _expert_body,
        grid=(n_tok, E),
        in_specs=[
            pl.BlockSpec((TOK_BLK, E), lambda i, e: (i, 0)),
            pl.BlockSpec((TOK_BLK, D), lambda i, e: (i, 0)),
            pl.BlockSpec((1, H, D), lambda i, e: (e, 0, 0)),
            pl.BlockSpec((1, 1, H), lambda i, e: (e, 0, 0)),
            pl.BlockSpec((1, D, H), lambda i, e: (e, 0, 0)),
            pl.BlockSpec((1, 1, D), lambda i, e: (e, 0, 0)),
        ],
        out_specs=pl.BlockSpec((TOK_BLK, D), lambda i, e: (i, 0)),
        out_shape=jax.ShapeDtypeStruct((S, D), jnp.float32),
    )(gate, xs, W1, b1.reshape(E, 1, H), W2, b2.reshape(E, 1, D))
    return out.reshape(x.shape)
